# Initial kernel scaffold; baseline (speedup 1.0000x reference)
import jax
import jax.numpy as jnp
from jax.experimental import pallas as pl


def kernel(x, W1, b1, Wl0, bl0, Wl1, bl1, Ww, bw, Wlast, blast, scaler, A_vals, edge_row, edge_col):
    raise NotImplementedError("write your pallas kernel here")


# speedup vs baseline: 2.1536x; 2.1536x over previous
import functools

import jax
import jax.numpy as jnp
from jax import lax
from jax.experimental import pallas as pl
from jax.experimental.pallas import tpu as pltpu
from jax.experimental.pallas import tpu_sc as plsc

N = 10000
E = 160000
DF = 256
H = 256
C = 64
T = 8

NC = 2
NS = 16
LANES = 16
HH = H // NC

K = 128
NCHUNK = 79
EPT = NCHUNK * K
EPAD = EPT * NS
RPT = 624
RREM = N - NS * RPT

ROWB = 1000
GRID = N // ROWB


def _linT(h, w_ref, b_ref):
    return lax.dot_general(h, w_ref[...], (((1,), (1,)), ((), ())),
                           preferred_element_type=jnp.float32) + b_ref[...]


def _mlp_body(x_ref, w1_ref, b1_ref, wl0_ref, bl0_ref, wl1_ref, bl1_ref,
              ww_ref, bw_ref, out_ref, z_ref):
    h = jnp.maximum(_linT(x_ref[...], w1_ref, b1_ref), 0.0)
    h = jnp.maximum(_linT(h, wl0_ref, bl0_ref), 0.0)
    h = jnp.maximum(_linT(h, wl1_ref, bl1_ref), 0.0)
    out_ref[...] = h
    z = _linT(h, ww_ref, bw_ref)
    z_ref[0] = z[:, :HH]
    z_ref[1] = z[:, HH:]


_mlp = pl.pallas_call(
    _mlp_body,
    grid=(GRID,),
    in_specs=[
        pl.BlockSpec((ROWB, DF), lambda i: (i, 0)),
        pl.BlockSpec((H, DF), lambda i: (0, 0)),
        pl.BlockSpec((1, H), lambda i: (0, 0)),
        pl.BlockSpec((H, H), lambda i: (0, 0)),
        pl.BlockSpec((1, H), lambda i: (0, 0)),
        pl.BlockSpec((H, H), lambda i: (0, 0)),
        pl.BlockSpec((1, H), lambda i: (0, 0)),
        pl.BlockSpec((H, H), lambda i: (0, 0)),
        pl.BlockSpec((1, H), lambda i: (0, 0)),
    ],
    out_specs=[
        pl.BlockSpec((ROWB, H), lambda i: (i, 0)),
        pl.BlockSpec((2, ROWB, HH), lambda i: (0, i, 0)),
    ],
    out_shape=[
        jax.ShapeDtypeStruct((N, H), jnp.float32),
        jax.ShapeDtypeStruct((2, N, HH), jnp.float32),
    ],
)


def _step_body(s_ref, o_in_ref, p_ref, ww_ref, bw_ref, out_ref, z_ref):
    s = s_ref[0]
    p = jnp.concatenate([p_ref[0], p_ref[1]], axis=1)
    o = o_in_ref[...] + s * jnp.maximum(p, 0.0)
    out_ref[...] = o
    z = _linT(o, ww_ref, bw_ref)
    z_ref[0] = z[:, :HH]
    z_ref[1] = z[:, HH:]


_step = pl.pallas_call(
    _step_body,
    grid=(GRID,),
    in_specs=[
        pl.BlockSpec(memory_space=pltpu.SMEM),
        pl.BlockSpec((ROWB, H), lambda i: (i, 0)),
        pl.BlockSpec((2, ROWB, HH), lambda i: (0, i, 0)),
        pl.BlockSpec((H, H), lambda i: (0, 0)),
        pl.BlockSpec((1, H), lambda i: (0, 0)),
    ],
    out_specs=[
        pl.BlockSpec((ROWB, H), lambda i: (i, 0)),
        pl.BlockSpec((2, ROWB, HH), lambda i: (0, i, 0)),
    ],
    out_shape=[
        jax.ShapeDtypeStruct((N, H), jnp.float32),
        jax.ShapeDtypeStruct((2, N, HH), jnp.float32),
    ],
)


def _final_body(s_ref, o_in_ref, p_ref, wl_ref, bl_ref, o_ref):
    s = s_ref[0]
    p = jnp.concatenate([p_ref[0], p_ref[1]], axis=1)
    o = o_in_ref[...] + s * jnp.maximum(p, 0.0)
    logits = _linT(o, wl_ref, bl_ref)
    m = jnp.max(logits, axis=1, keepdims=True)
    ex = jnp.exp(logits - m)
    lse = jnp.log(jnp.sum(ex, axis=1, keepdims=True))
    o_ref[...] = logits - m - lse


_final = pl.pallas_call(
    _final_body,
    grid=(GRID,),
    in_specs=[
        pl.BlockSpec(memory_space=pltpu.SMEM),
        pl.BlockSpec((ROWB, H), lambda i: (i, 0)),
        pl.BlockSpec((2, ROWB, HH), lambda i: (0, i, 0)),
        pl.BlockSpec((C, H), lambda i: (0, 0)),
        pl.BlockSpec((1, C), lambda i: (0, 0)),
    ],
    out_specs=pl.BlockSpec((ROWB, C), lambda i: (i, 0)),
    out_shape=jax.ShapeDtypeStruct((N, C), jnp.float32),
)


def _spmm_body(z_hbm, edge_hbm, val_hbm, out_hbm, ebuf, vbuf, gbuf, acc,
               esem, vsem, gsem):
    c = lax.axis_index("c")
    s = lax.axis_index("s")

    zv = jnp.zeros((LANES,), jnp.float32)

    def zrow(r, _):
        for f in range(HH // LANES):
            gbuf[0, r, pl.ds(f * LANES, LANES)] = zv
        return 0
    lax.fori_loop(0, K, zrow, 0)

    base = s * RPT
    for kk in range(RPT // K):
        pltpu.sync_copy(gbuf.at[0], acc.at[pl.ds(base + kk * K, K)])
    rem = RPT % K
    if rem:
        pltpu.sync_copy(gbuf.at[0, pl.ds(0, rem)],
                        acc.at[pl.ds(base + (RPT // K) * K, rem)])

    @pl.when(s == NS - 1)
    def _():
        pltpu.sync_copy(gbuf.at[0, pl.ds(0, RREM)],
                        acc.at[pl.ds(NS * RPT, RREM)])
    plsc.subcore_barrier()

    def start_edges(j, b):
        pltpu.async_copy(edge_hbm.at[s, j], ebuf.at[b], esem.at[b])
        pltpu.async_copy(val_hbm.at[s, j], vbuf.at[b], vsem.at[b])

    def wait_edges(j, b):
        pltpu.make_async_copy(edge_hbm.at[s, j], ebuf.at[b],
                              esem.at[b]).wait()
        pltpu.make_async_copy(val_hbm.at[s, j], vbuf.at[b],
                              vsem.at[b]).wait()

    def start_gather(j, b):
        pltpu.async_copy(z_hbm.at[ebuf.at[b, c]], gbuf.at[b], gsem.at[b])

    def wait_gather(j, b):
        pltpu.make_async_copy(z_hbm.at[ebuf.at[b, c]], gbuf.at[b],
                              gsem.at[b]).wait()

    start_edges(0, 0)
    start_edges(1, 1)
    wait_edges(0, 0)
    start_gather(0, 0)

    def chunk(j, _):
        b = j % 2
        nb = (j + 1) % 2

        @pl.when(j + 1 < NCHUNK)
        def _():
            wait_edges(j + 1, nb)
            start_gather(j + 1, nb)

        wait_gather(j, b)

        def edge_group(eg, _):
            vals16 = vbuf[b, pl.ds(eg * LANES, LANES)]
            for el in range(LANES):
                vb = lax.gather(
                    vals16, jnp.full((LANES, 1), el, jnp.int32),
                    lax.GatherDimensionNumbers(
                        offset_dims=(), collapsed_slice_dims=(0,),
                        start_index_map=(0,)),
                    (1,), mode=lax.GatherScatterMode.PROMISE_IN_BOUNDS)
                e = eg * LANES + el
                for f in range(HH // LANES):
                    sl = pl.ds(f * LANES, LANES)
                    gbuf[b, e, sl] = gbuf[b, e, sl] * vb
            return 0
        lax.fori_loop(0, K // LANES, edge_group, 0)

        pltpu.sync_copy(gbuf.at[b], acc.at[ebuf.at[b, 2]], add=True)

        @pl.when(j + 2 < NCHUNK)
        def _():
            start_edges(j + 2, b)
        return 0
    lax.fori_loop(0, NCHUNK, chunk, 0)

    plsc.subcore_barrier()
    pltpu.sync_copy(acc.at[pl.ds(s * RPT, RPT)],
                    out_hbm.at[pl.ds(c * N + s * RPT, RPT)])

    @pl.when(s == NS - 1)
    def _():
        pltpu.sync_copy(acc.at[pl.ds(NS * RPT, RREM)],
                        out_hbm.at[pl.ds(c * N + NS * RPT, RREM)])


_spmm = pl.kernel(
    _spmm_body,
    out_type=jax.ShapeDtypeStruct((2 * N, HH), jnp.float32),
    mesh=plsc.VectorSubcoreMesh(core_axis_name="c", subcore_axis_name="s",
                                num_cores=NC, num_subcores=NS),
    scratch_types=[
        pltpu.VMEM((2, 3, K), jnp.int32),
        pltpu.VMEM((2, K), jnp.float32),
        pltpu.VMEM((2, K, HH), jnp.float32),
        pltpu.VMEM_SHARED((N, HH), jnp.float32),
        pltpu.SemaphoreType.DMA((2,)),
        pltpu.SemaphoreType.DMA((2,)),
        pltpu.SemaphoreType.DMA((2,)),
    ],
)


def kernel(x, W1, b1, Wl0, bl0, Wl1, bl1, Ww, bw, Wlast, blast, scaler,
           A_vals, edge_row, edge_col):
    b1r = b1.reshape(1, H)
    bl0r = bl0.reshape(1, H)
    bl1r = bl1.reshape(1, H)
    bwr = bw.reshape(1, H)
    blastr = blast.reshape(1, C)

    pad = EPAD - E
    colp = jnp.concatenate(
        [edge_col, jnp.zeros((pad,), jnp.int32)]).reshape(NS, NCHUNK, K)
    rowp = jnp.concatenate(
        [edge_row, jnp.zeros((pad,), jnp.int32)]).reshape(NS, NCHUNK, K)
    valp = jnp.concatenate(
        [A_vals, jnp.zeros((pad,), jnp.float32)]).reshape(NS, NCHUNK, K)
    edges = jnp.stack([colp, colp + N, rowp], axis=2)

    out, z = _mlp(x, W1, b1r, Wl0, bl0r, Wl1, bl1r, Ww, bwr)
    zf = z.reshape(2 * N, HH)
    for t in range(T):
        p = _spmm(zf, edges, valp)
        st = scaler[t]
        if t < T - 1:
            out, z = _step(st, out, p.reshape(2, N, HH), Ww, bwr)
            zf = z.reshape(2 * N, HH)
        else:
            res = _final(st, out, p.reshape(2, N, HH), Wlast, blastr)
    return res


# speedup vs baseline: 3.8159x; 1.7719x over previous
import functools

import jax
import jax.numpy as jnp
from jax import lax
from jax.experimental import pallas as pl
from jax.experimental.pallas import tpu as pltpu
from jax.experimental.pallas import tpu_sc as plsc

N = 10000
E = 160000
DF = 256
H = 256
C = 64
T = 8

NC = 2
NS = 16
LANES = 16
HH = H // NC

K = 128
NCHUNK = 79
EPT = NCHUNK * K
EPAD = EPT * NS
RPT = 624
RREM = N - NS * RPT

ROWB = 1000
GRID = N // ROWB


def _linT(h, w_ref, b_ref):
    return lax.dot_general(h, w_ref[...], (((1,), (1,)), ((), ())),
                           preferred_element_type=jnp.float32) + b_ref[...]


def _mlp_body(x_ref, w1_ref, b1_ref, wl0_ref, bl0_ref, wl1_ref, bl1_ref,
              ww_ref, bw_ref, out_ref, z_ref):
    h = jnp.maximum(_linT(x_ref[...], w1_ref, b1_ref), 0.0)
    h = jnp.maximum(_linT(h, wl0_ref, bl0_ref), 0.0)
    h = jnp.maximum(_linT(h, wl1_ref, bl1_ref), 0.0)
    out_ref[...] = h
    z = _linT(h, ww_ref, bw_ref)
    z_ref[0] = z[:, :HH]
    z_ref[1] = z[:, HH:]


_mlp = pl.pallas_call(
    _mlp_body,
    grid=(GRID,),
    in_specs=[
        pl.BlockSpec((ROWB, DF), lambda i: (i, 0)),
        pl.BlockSpec((H, DF), lambda i: (0, 0)),
        pl.BlockSpec((1, H), lambda i: (0, 0)),
        pl.BlockSpec((H, H), lambda i: (0, 0)),
        pl.BlockSpec((1, H), lambda i: (0, 0)),
        pl.BlockSpec((H, H), lambda i: (0, 0)),
        pl.BlockSpec((1, H), lambda i: (0, 0)),
        pl.BlockSpec((H, H), lambda i: (0, 0)),
        pl.BlockSpec((1, H), lambda i: (0, 0)),
    ],
    out_specs=[
        pl.BlockSpec((ROWB, H), lambda i: (i, 0)),
        pl.BlockSpec((2, ROWB, HH), lambda i: (0, i, 0)),
    ],
    out_shape=[
        jax.ShapeDtypeStruct((N, H), jnp.float32),
        jax.ShapeDtypeStruct((2, N, HH), jnp.float32),
    ],
)


def _step_body(s_ref, o_in_ref, p_ref, ww_ref, bw_ref, out_ref, z_ref):
    s = s_ref[0]
    p = jnp.concatenate([p_ref[0], p_ref[1]], axis=1)
    o = o_in_ref[...] + s * jnp.maximum(p, 0.0)
    out_ref[...] = o
    z = _linT(o, ww_ref, bw_ref)
    z_ref[0] = z[:, :HH]
    z_ref[1] = z[:, HH:]


_step = pl.pallas_call(
    _step_body,
    grid=(GRID,),
    in_specs=[
        pl.BlockSpec(memory_space=pltpu.SMEM),
        pl.BlockSpec((ROWB, H), lambda i: (i, 0)),
        pl.BlockSpec((2, ROWB, HH), lambda i: (0, i, 0)),
        pl.BlockSpec((H, H), lambda i: (0, 0)),
        pl.BlockSpec((1, H), lambda i: (0, 0)),
    ],
    out_specs=[
        pl.BlockSpec((ROWB, H), lambda i: (i, 0)),
        pl.BlockSpec((2, ROWB, HH), lambda i: (0, i, 0)),
    ],
    out_shape=[
        jax.ShapeDtypeStruct((N, H), jnp.float32),
        jax.ShapeDtypeStruct((2, N, HH), jnp.float32),
    ],
)


def _final_body(s_ref, o_in_ref, p_ref, wl_ref, bl_ref, o_ref):
    s = s_ref[0]
    p = jnp.concatenate([p_ref[0], p_ref[1]], axis=1)
    o = o_in_ref[...] + s * jnp.maximum(p, 0.0)
    logits = _linT(o, wl_ref, bl_ref)
    m = jnp.max(logits, axis=1, keepdims=True)
    ex = jnp.exp(logits - m)
    lse = jnp.log(jnp.sum(ex, axis=1, keepdims=True))
    o_ref[...] = logits - m - lse


_final = pl.pallas_call(
    _final_body,
    grid=(GRID,),
    in_specs=[
        pl.BlockSpec(memory_space=pltpu.SMEM),
        pl.BlockSpec((ROWB, H), lambda i: (i, 0)),
        pl.BlockSpec((2, ROWB, HH), lambda i: (0, i, 0)),
        pl.BlockSpec((C, H), lambda i: (0, 0)),
        pl.BlockSpec((1, C), lambda i: (0, 0)),
    ],
    out_specs=pl.BlockSpec((ROWB, C), lambda i: (i, 0)),
    out_shape=jax.ShapeDtypeStruct((N, C), jnp.float32),
)


def _spmm_body(z_hbm, edge_hbm, val_hbm, out_hbm, ebuf, vbuf, gbuf, sbuf,
               acc, esem, vsem, gsem):
    c = lax.axis_index("c")
    s = lax.axis_index("s")

    zv = jnp.zeros((LANES,), jnp.float32)

    def zrow(r, _):
        for f in range(HH // LANES):
            gbuf[0, r, pl.ds(f * LANES, LANES)] = zv
        return 0
    lax.fori_loop(0, K, zrow, 0)

    base = s * RPT
    for kk in range(RPT // K):
        pltpu.sync_copy(gbuf.at[0], acc.at[pl.ds(base + kk * K, K)])
    rem = RPT % K
    if rem:
        pltpu.sync_copy(gbuf.at[0, pl.ds(0, rem)],
                        acc.at[pl.ds(base + (RPT // K) * K, rem)])

    @pl.when(s == NS - 1)
    def _():
        pltpu.sync_copy(gbuf.at[0, pl.ds(0, RREM)],
                        acc.at[pl.ds(NS * RPT, RREM)])
    plsc.subcore_barrier()

    def start_edges(j, b):
        pltpu.async_copy(edge_hbm.at[s, j], ebuf.at[b], esem.at[b])
        pltpu.async_copy(val_hbm.at[s, j], vbuf.at[b], vsem.at[b])

    def wait_edges(j, b):
        pltpu.make_async_copy(edge_hbm.at[s, j], ebuf.at[b],
                              esem.at[b]).wait()
        pltpu.make_async_copy(val_hbm.at[s, j], vbuf.at[b],
                              vsem.at[b]).wait()

    def start_gather(j, b):
        pltpu.async_copy(z_hbm.at[ebuf.at[b, c]], gbuf.at[b], gsem.at[b])

    def wait_gather(j, b):
        pltpu.make_async_copy(z_hbm.at[ebuf.at[b, c]], gbuf.at[b],
                              gsem.at[b]).wait()

    start_edges(0, 0)
    start_edges(1, 1)
    wait_edges(0, 0)
    start_gather(0, 0)

    def chunk(j, _):
        b = j % 2
        nb = (j + 1) % 2

        @pl.when(j + 1 < NCHUNK)
        def _():
            wait_edges(j + 1, nb)
            start_gather(j + 1, nb)

        wait_gather(j, b)

        def edge_group(eg, _):
            vals16 = vbuf[b, pl.ds(eg * LANES, LANES)]
            for el in range(LANES):
                vb = lax.gather(
                    vals16, jnp.full((LANES, 1), el, jnp.int32),
                    lax.GatherDimensionNumbers(
                        offset_dims=(), collapsed_slice_dims=(0,),
                        start_index_map=(0,)),
                    (1,), mode=lax.GatherScatterMode.PROMISE_IN_BOUNDS)
                e = eg * LANES + el
                xs = [gbuf[b, e, pl.ds(f * LANES, LANES)]
                      for f in range(HH // LANES)]
                ys = [x * vb for x in xs]
                for f in range(HH // LANES):
                    sbuf[e, pl.ds(f * LANES, LANES)] = ys[f]
            return 0
        lax.fori_loop(0, K // LANES, edge_group, 0)

        pltpu.sync_copy(sbuf, acc.at[ebuf.at[b, 2]], add=True)

        @pl.when(j + 2 < NCHUNK)
        def _():
            start_edges(j + 2, b)
        return 0
    lax.fori_loop(0, NCHUNK, chunk, 0)

    plsc.subcore_barrier()
    pltpu.sync_copy(acc.at[pl.ds(s * RPT, RPT)],
                    out_hbm.at[pl.ds(c * N + s * RPT, RPT)])

    @pl.when(s == NS - 1)
    def _():
        pltpu.sync_copy(acc.at[pl.ds(NS * RPT, RREM)],
                        out_hbm.at[pl.ds(c * N + NS * RPT, RREM)])


_spmm = pl.kernel(
    _spmm_body,
    out_type=jax.ShapeDtypeStruct((2 * N, HH), jnp.float32),
    mesh=plsc.VectorSubcoreMesh(core_axis_name="c", subcore_axis_name="s",
                                num_cores=NC, num_subcores=NS),
    scratch_types=[
        pltpu.VMEM((2, 3, K), jnp.int32),
        pltpu.VMEM((2, K), jnp.float32),
        pltpu.VMEM((2, K, HH), jnp.float32),
        pltpu.VMEM((K, HH), jnp.float32),
        pltpu.VMEM_SHARED((N, HH), jnp.float32),
        pltpu.SemaphoreType.DMA((2,)),
        pltpu.SemaphoreType.DMA((2,)),
        pltpu.SemaphoreType.DMA((2,)),
    ],
)


def kernel(x, W1, b1, Wl0, bl0, Wl1, bl1, Ww, bw, Wlast, blast, scaler,
           A_vals, edge_row, edge_col):
    b1r = b1.reshape(1, H)
    bl0r = bl0.reshape(1, H)
    bl1r = bl1.reshape(1, H)
    bwr = bw.reshape(1, H)
    blastr = blast.reshape(1, C)

    pad = EPAD - E
    colp = jnp.concatenate(
        [edge_col, jnp.zeros((pad,), jnp.int32)]).reshape(NS, NCHUNK, K)
    rowp = jnp.concatenate(
        [edge_row, jnp.zeros((pad,), jnp.int32)]).reshape(NS, NCHUNK, K)
    valp = jnp.concatenate(
        [A_vals, jnp.zeros((pad,), jnp.float32)]).reshape(NS, NCHUNK, K)
    edges = jnp.stack([colp, colp + N, rowp], axis=2)

    out, z = _mlp(x, W1, b1r, Wl0, bl0r, Wl1, bl1r, Ww, bwr)
    zf = z.reshape(2 * N, HH)
    for t in range(T):
        p = _spmm(zf, edges, valp)
        st = scaler[t]
        if t < T - 1:
            out, z = _step(st, out, p.reshape(2, N, HH), Ww, bwr)
            zf = z.reshape(2 * N, HH)
        else:
            res = _final(st, out, p.reshape(2, N, HH), Wlast, blastr)
    return res


# speedup vs baseline: 6.2873x; 1.6476x over previous
import functools

import jax
import jax.numpy as jnp
from jax import lax
from jax.experimental import pallas as pl
from jax.experimental.pallas import tpu as pltpu
from jax.experimental.pallas import tpu_sc as plsc

N = 10000
E = 160000
DF = 256
H = 256
C = 64
T = 8

NC = 2
NS = 16
LANES = 16
HH = H // NC

K = 80
NCHUNK = 125
EPT = NCHUNK * K
EPAD = EPT * NS
RPT = 624
RREM = N - NS * RPT

ROWB = 1000
GRID = N // ROWB


def _linT(h, w_ref, b_ref):
    return lax.dot_general(h, w_ref[...], (((1,), (1,)), ((), ())),
                           preferred_element_type=jnp.float32) + b_ref[...]


def _mlp_body(x_ref, w1_ref, b1_ref, wl0_ref, bl0_ref, wl1_ref, bl1_ref,
              ww_ref, bw_ref, out_ref, z_ref):
    h = jnp.maximum(_linT(x_ref[...], w1_ref, b1_ref), 0.0)
    h = jnp.maximum(_linT(h, wl0_ref, bl0_ref), 0.0)
    h = jnp.maximum(_linT(h, wl1_ref, bl1_ref), 0.0)
    out_ref[...] = h
    z = _linT(h, ww_ref, bw_ref)
    z_ref[0] = z[:, :HH]
    z_ref[1] = z[:, HH:]


_mlp = pl.pallas_call(
    _mlp_body,
    grid=(GRID,),
    in_specs=[
        pl.BlockSpec((ROWB, DF), lambda i: (i, 0)),
        pl.BlockSpec((H, DF), lambda i: (0, 0)),
        pl.BlockSpec((1, H), lambda i: (0, 0)),
        pl.BlockSpec((H, H), lambda i: (0, 0)),
        pl.BlockSpec((1, H), lambda i: (0, 0)),
        pl.BlockSpec((H, H), lambda i: (0, 0)),
        pl.BlockSpec((1, H), lambda i: (0, 0)),
        pl.BlockSpec((H, H), lambda i: (0, 0)),
        pl.BlockSpec((1, H), lambda i: (0, 0)),
    ],
    out_specs=[
        pl.BlockSpec((ROWB, H), lambda i: (i, 0)),
        pl.BlockSpec((2, ROWB, HH), lambda i: (0, i, 0)),
    ],
    out_shape=[
        jax.ShapeDtypeStruct((N, H), jnp.float32),
        jax.ShapeDtypeStruct((2, N, HH), jnp.float32),
    ],
)


def _step_body(s_ref, o_in_ref, p_ref, ww_ref, bw_ref, out_ref, z_ref):
    s = s_ref[0]
    p = jnp.concatenate([p_ref[0], p_ref[1]], axis=1)
    o = o_in_ref[...] + s * jnp.maximum(p, 0.0)
    out_ref[...] = o
    z = _linT(o, ww_ref, bw_ref)
    z_ref[0] = z[:, :HH]
    z_ref[1] = z[:, HH:]


_step = pl.pallas_call(
    _step_body,
    grid=(GRID,),
    in_specs=[
        pl.BlockSpec(memory_space=pltpu.SMEM),
        pl.BlockSpec((ROWB, H), lambda i: (i, 0)),
        pl.BlockSpec((2, ROWB, HH), lambda i: (0, i, 0)),
        pl.BlockSpec((H, H), lambda i: (0, 0)),
        pl.BlockSpec((1, H), lambda i: (0, 0)),
    ],
    out_specs=[
        pl.BlockSpec((ROWB, H), lambda i: (i, 0)),
        pl.BlockSpec((2, ROWB, HH), lambda i: (0, i, 0)),
    ],
    out_shape=[
        jax.ShapeDtypeStruct((N, H), jnp.float32),
        jax.ShapeDtypeStruct((2, N, HH), jnp.float32),
    ],
)


def _final_body(s_ref, o_in_ref, p_ref, wl_ref, bl_ref, o_ref):
    s = s_ref[0]
    p = jnp.concatenate([p_ref[0], p_ref[1]], axis=1)
    o = o_in_ref[...] + s * jnp.maximum(p, 0.0)
    logits = _linT(o, wl_ref, bl_ref)
    m = jnp.max(logits, axis=1, keepdims=True)
    ex = jnp.exp(logits - m)
    lse = jnp.log(jnp.sum(ex, axis=1, keepdims=True))
    o_ref[...] = logits - m - lse


_final = pl.pallas_call(
    _final_body,
    grid=(GRID,),
    in_specs=[
        pl.BlockSpec(memory_space=pltpu.SMEM),
        pl.BlockSpec((ROWB, H), lambda i: (i, 0)),
        pl.BlockSpec((2, ROWB, HH), lambda i: (0, i, 0)),
        pl.BlockSpec((C, H), lambda i: (0, 0)),
        pl.BlockSpec((1, C), lambda i: (0, 0)),
    ],
    out_specs=pl.BlockSpec((ROWB, C), lambda i: (i, 0)),
    out_shape=jax.ShapeDtypeStruct((N, C), jnp.float32),
)


def _spmm_body(z_hbm, edge_hbm, val_hbm, out_hbm, ebuf, vbuf, gbuf, sbuf,
               acc, esem, vsem, gsem, ssem):
    c = lax.axis_index("c")
    s = lax.axis_index("s")

    zv = jnp.zeros((LANES,), jnp.float32)

    def zrow(r, _):
        for f in range(HH // LANES):
            gbuf[0, r, pl.ds(f * LANES, LANES)] = zv
        return 0
    lax.fori_loop(0, K, zrow, 0)

    base = s * RPT
    for kk in range(RPT // K):
        pltpu.sync_copy(gbuf.at[0], acc.at[pl.ds(base + kk * K, K)])
    rem = RPT % K
    if rem:
        pltpu.sync_copy(gbuf.at[0, pl.ds(0, rem)],
                        acc.at[pl.ds(base + (RPT // K) * K, rem)])

    @pl.when(s == NS - 1)
    def _():
        pltpu.sync_copy(gbuf.at[0, pl.ds(0, RREM)],
                        acc.at[pl.ds(NS * RPT, RREM)])
    plsc.subcore_barrier()

    def start_edges(j):
        b = j % 4
        pltpu.async_copy(edge_hbm.at[s, j], ebuf.at[b], esem.at[b])
        pltpu.async_copy(val_hbm.at[s, j], vbuf.at[b], vsem.at[b])

    def wait_edges(j):
        b = j % 4
        pltpu.make_async_copy(edge_hbm.at[s, j], ebuf.at[b],
                              esem.at[b]).wait()
        pltpu.make_async_copy(val_hbm.at[s, j], vbuf.at[b],
                              vsem.at[b]).wait()

    def start_gather(j):
        pltpu.async_copy(z_hbm.at[ebuf.at[j % 4, c]], gbuf.at[j % 2],
                         gsem.at[j % 2])

    def wait_gather(j):
        pltpu.make_async_copy(z_hbm.at[ebuf.at[j % 4, c]], gbuf.at[j % 2],
                              gsem.at[j % 2]).wait()

    def start_scatter(j):
        pltpu.async_copy(sbuf.at[j % 2], acc.at[ebuf.at[j % 4, 2]],
                         ssem.at[j % 2], add=True)

    def wait_scatter(j):
        pltpu.make_async_copy(sbuf.at[j % 2], acc.at[ebuf.at[j % 4, 2]],
                              ssem.at[j % 2]).wait()

    start_edges(0)
    start_edges(1)
    start_edges(2)
    wait_edges(0)
    start_gather(0)

    def chunk(j, _):
        g = j % 2
        e4 = j % 4
        wait_gather(j)

        @pl.when(j + 1 < NCHUNK)
        def _():
            wait_edges(j + 1)
            start_gather(j + 1)

        def edge_group(eg, _):
            vals16 = vbuf[e4, pl.ds(eg * LANES, LANES)]
            for el in range(LANES):
                vb = lax.gather(
                    vals16, jnp.full((LANES, 1), el, jnp.int32),
                    lax.GatherDimensionNumbers(
                        offset_dims=(), collapsed_slice_dims=(0,),
                        start_index_map=(0,)),
                    (1,), mode=lax.GatherScatterMode.PROMISE_IN_BOUNDS)
                e = eg * LANES + el
                xs = [gbuf[g, e, pl.ds(f * LANES, LANES)]
                      for f in range(HH // LANES)]
                ys = [x * vb for x in xs]
                for f in range(HH // LANES):
                    sbuf[g, e, pl.ds(f * LANES, LANES)] = ys[f]
            return 0
        lax.fori_loop(0, K // LANES, edge_group, 0)

        start_scatter(j)

        @pl.when(j >= 1)
        def _():
            wait_scatter(j - 1)

        @pl.when(j + 3 < NCHUNK)
        def _():
            start_edges(j + 3)
        return 0
    lax.fori_loop(0, NCHUNK, chunk, 0)

    wait_scatter(NCHUNK - 1)
    plsc.subcore_barrier()
    pltpu.sync_copy(acc.at[pl.ds(s * RPT, RPT)],
                    out_hbm.at[pl.ds(c * N + s * RPT, RPT)])

    @pl.when(s == NS - 1)
    def _():
        pltpu.sync_copy(acc.at[pl.ds(NS * RPT, RREM)],
                        out_hbm.at[pl.ds(c * N + NS * RPT, RREM)])


_spmm = pl.kernel(
    _spmm_body,
    out_type=jax.ShapeDtypeStruct((2 * N, HH), jnp.float32),
    mesh=plsc.VectorSubcoreMesh(core_axis_name="c", subcore_axis_name="s",
                                num_cores=NC, num_subcores=NS),
    scratch_types=[
        pltpu.VMEM((4, 3, K), jnp.int32),
        pltpu.VMEM((4, K), jnp.float32),
        pltpu.VMEM((2, K, HH), jnp.float32),
        pltpu.VMEM((2, K, HH), jnp.float32),
        pltpu.VMEM_SHARED((N, HH), jnp.float32),
        pltpu.SemaphoreType.DMA((4,)),
        pltpu.SemaphoreType.DMA((4,)),
        pltpu.SemaphoreType.DMA((2,)),
        pltpu.SemaphoreType.DMA((2,)),
    ],
)


def kernel(x, W1, b1, Wl0, bl0, Wl1, bl1, Ww, bw, Wlast, blast, scaler,
           A_vals, edge_row, edge_col):
    b1r = b1.reshape(1, H)
    bl0r = bl0.reshape(1, H)
    bl1r = bl1.reshape(1, H)
    bwr = bw.reshape(1, H)
    blastr = blast.reshape(1, C)

    colp = edge_col.reshape(NS, NCHUNK, K)
    rowp = edge_row.reshape(NS, NCHUNK, K)
    valp = A_vals.reshape(NS, NCHUNK, K)
    edges = jnp.stack([colp, colp + N, rowp], axis=2)

    out, z = _mlp(x, W1, b1r, Wl0, bl0r, Wl1, bl1r, Ww, bwr)
    zf = z.reshape(2 * N, HH)
    for t in range(T):
        p = _spmm(zf, edges, valp)
        st = scaler[t]
        if t < T - 1:
            out, z = _step(st, out, p.reshape(2, N, HH), Ww, bwr)
            zf = z.reshape(2 * N, HH)
        else:
            res = _final(st, out, p.reshape(2, N, HH), Wlast, blastr)
    return res
